# Initial kernel scaffold; baseline (speedup 1.0000x reference)
import jax
import jax.numpy as jnp
from jax.experimental import pallas as pl


def kernel(gps_idx, table):
    raise NotImplementedError("write your pallas kernel here")


# speedup vs baseline: 1.8724x; 1.8724x over previous
import functools

import jax
import jax.numpy as jnp
from jax import lax
from jax.experimental import pallas as pl
from jax.experimental.pallas import tpu as pltpu
from jax.experimental.pallas import tpu_sc as plsc

NUM_GPS = 1000000
EMBED_DIM = 64
BATCH = 16384
HIST = 50

B = BATCH * HIST
NC = 2
NS = 16
NW = NC * NS
BPW = B // NW
SEG = 128
NSEG_W = BPW // SEG
CH = 512
NSEG_C = CH // SEG
NCH = BPW // CH


def _gather_body(idx_hbm, table_hbm, out_hbm, idx_v, rows_v, sg0, sg1, ss0, ss1):
    wid = lax.axis_index("s") * NC + lax.axis_index("c")
    sem_g = (sg0, sg1)
    sem_s = (ss0, ss1)

    pltpu.sync_copy(idx_hbm.at[wid], idx_v)

    def issue_gathers(c, b):
        for k in range(NSEG_C):
            pltpu.async_copy(
                table_hbm.at[idx_v.at[c * NSEG_C + k]],
                rows_v.at[b].at[pl.ds(k * SEG, SEG)],
                sem_g[b],
            )

    def wait_gathers(b):
        pltpu.make_async_copy(
            table_hbm.at[pl.ds(0, CH)], rows_v.at[b], sem_g[b]
        ).wait()

    def wait_store(b):
        pltpu.make_async_copy(
            rows_v.at[b], table_hbm.at[pl.ds(0, CH)], sem_s[b]
        ).wait()

    issue_gathers(0, 0)
    issue_gathers(1, 1)

    def loop_body(g, _):
        for b in range(2):
            c = 2 * g + b
            wait_gathers(b)
            pltpu.async_copy(rows_v.at[b], out_hbm.at[wid, c], sem_s[b])

            @pl.when(c + 2 < NCH)
            def _():
                wait_store(b)
                issue_gathers(c + 2, b)

        return _

    lax.fori_loop(0, NCH // 2, loop_body, None)

    wait_store(0)
    wait_store(1)


def kernel(gps_idx, table):
    idx_flat = gps_idx.reshape(NW, NSEG_W, SEG).astype(jnp.int32)

    mesh = plsc.VectorSubcoreMesh(core_axis_name="c", subcore_axis_name="s")
    out = pl.kernel(
        _gather_body,
        mesh=mesh,
        out_type=jax.ShapeDtypeStruct((NW, NCH, CH, EMBED_DIM), jnp.float32),
        scratch_types=[
            pltpu.VMEM((NSEG_W, SEG), jnp.int32),
            pltpu.VMEM((2, CH, EMBED_DIM), jnp.float32),
            pltpu.SemaphoreType.DMA,
            pltpu.SemaphoreType.DMA,
            pltpu.SemaphoreType.DMA,
            pltpu.SemaphoreType.DMA,
        ],
        compiler_params=pltpu.CompilerParams(use_tc_tiling_on_sc=False),
    )(idx_flat, table)
    return out.reshape(BATCH, HIST, EMBED_DIM)
